# baseline (speedup 1.0000x reference)
import functools

import jax
import jax.numpy as jnp
import numpy as np
from jax import lax
from jax.experimental import pallas as pl
from jax.experimental.pallas import tpu as pltpu
from jax.experimental.pallas import tpu_sc as plsc

NCELL = 2304
NB = 4096
NA = 6
NSC = 2
NTEC = 16
CPT = NCELL // NTEC
UPS = NB // NSC
UPT = UPS // NTEC
NCH = UPS // 64
FMIN = float(jnp.finfo(jnp.float32).min)

_cols = np.arange(NB)
_c, _r = _cols // 64, _cols % 64
_g, _w = _r // 16, _r % 16
_UNIT_OF_COL = _c * 64 + 4 * _w + _g
_INV = np.argsort(_UNIT_OF_COL)


def _tec_body(mask_hbm, starts_hbm, logit_hbm, out_hbm,
              mask_v, st_v, logit_v, acc_v):
    c = lax.axis_index("c")
    s = lax.axis_index("s")
    r0 = s * CPT
    u0 = c * UPS

    pltpu.sync_copy(mask_hbm.at[pl.ds(r0, CPT), pl.ds(c * (UPS // 4), UPS // 4)], mask_v)
    pltpu.sync_copy(logit_hbm.at[pl.ds(r0, CPT)], logit_v)
    pltpu.sync_copy(starts_hbm.at[s], st_v)

    st = st_v[0, :]
    bounds = [st[i] for i in range(NA + 1)]

    def chunk_body(ch, _):
        base = ch * 16
        for a in range(NA):
            lo = bounds[a]
            hi = bounds[a + 1]

            def cell(j, carry):
                sums, cnts = carry
                mw = mask_v[j, pl.ds(base, 16)]
                lg = logit_v[j, :]
                new_s, new_c = [], []
                for g in range(4):
                    bit = (mw >> (8 * g)) & 1
                    f = bit.astype(jnp.float32)
                    new_s.append(sums[g] + f * lg)
                    new_c.append(cnts[g] + bit)
                return tuple(new_s), tuple(new_c)

            z = jnp.zeros((16,), jnp.float32)
            zi = jnp.zeros((16,), jnp.int32)
            sums, cnts = lax.fori_loop(
                lo, hi, cell, ((z, z, z, z), (zi, zi, zi, zi)))
            for g in range(4):
                off = ch * 64 + g * 16
                acc_v[a, pl.ds(off, 16)] = sums[g]
                acc_v[NA + a, pl.ds(off, 16)] = cnts[g].astype(jnp.float32)
        return 0

    lax.fori_loop(0, NCH, chunk_body, 0)

    pltpu.sync_copy(acc_v, out_hbm.at[s, :, pl.ds(u0, UPS)])


def _post_body(part_ref, logits_ref, conv_ref, out_ref):
    del logits_ref, conv_ref
    acc = jnp.sum(part_ref[...], axis=0)
    sums = acc[:NA, :]
    counts = acc[NA:, :]
    total = jnp.sum(counts, axis=0, keepdims=True)
    row = jax.lax.broadcasted_iota(jnp.int32, sums.shape, 0)
    scaled = jnp.where(row == NA - 1, sums * (1.0 / 225.0), sums)
    o = jnp.where(counts > 0.5, scaled, FMIN)
    out_ref[...] = jnp.where((total < 0.5) & (row == 0), 1.0, o)


def kernel(logits, monoaction_mask, monofield_base_converter):
    conv = monofield_base_converter.reshape(NCELL)
    local = conv.reshape(NTEC, CPT)
    order = jnp.argsort(local, axis=1, stable=True)
    perm = (order + jnp.arange(NTEC, dtype=order.dtype)[:, None] * CPT
            ).reshape(NCELL)
    seg = jnp.sum(local[:, None, :] < jnp.arange(NA, dtype=jnp.int32)[None, :, None],
                  axis=-1).astype(jnp.int32)
    starts = jnp.concatenate(
        [seg, jnp.full((NTEC, 1), CPT, jnp.int32),
         jnp.zeros((NTEC, 16 - NA - 1), jnp.int32)], axis=1)

    mask_lin = (monoaction_mask.transpose(1, 2, 0)
                .reshape(NCELL, NB).view(jnp.int8))
    mask_sorted = mask_lin[perm].view(jnp.int32)
    logit_b = jnp.broadcast_to(
        logits.reshape(NCELL)[perm][:, None], (NCELL, 16))

    mesh = plsc.VectorSubcoreMesh(core_axis_name="c", subcore_axis_name="s")
    k = functools.partial(
        pl.kernel,
        mesh=mesh,
        out_type=jax.ShapeDtypeStruct((NTEC, 2 * NA, NB), jnp.float32),
        scratch_types=[
            pltpu.VMEM((CPT, UPS // 4), jnp.int32),
            pltpu.VMEM((1, 16), jnp.int32),
            pltpu.VMEM((CPT, 16), jnp.float32),
            pltpu.VMEM((2 * NA, UPS), jnp.float32),
        ],
    )(_tec_body)
    part = k(mask_sorted, starts.reshape(NTEC, 1, 16), logit_b)
    out_cols = pl.pallas_call(
        _post_body,
        in_specs=[
            pl.BlockSpec((NTEC, 2 * NA, NB), lambda: (0, 0, 0)),
            pl.BlockSpec((1, NCELL), lambda: (0, 0)),
            pl.BlockSpec((1, NCELL), lambda: (0, 0)),
        ],
        out_specs=pl.BlockSpec((NA, NB), lambda: (0, 0)),
        out_shape=jax.ShapeDtypeStruct((NA, NB), jnp.float32),
    )(part, logits.reshape(1, NCELL),
      monofield_base_converter.reshape(1, NCELL))
    return out_cols[:, _INV].T


# speedup vs baseline: 14.0554x; 14.0554x over previous
import jax
import jax.numpy as jnp
from jax.experimental import pallas as pl
from jax.experimental.pallas import tpu as pltpu

GRID = (48, 48)
NCELL = GRID[0] * GRID[1]
NA = 6
NB = 4096
BLK = 2048
FMIN = jnp.finfo(jnp.float32).min


def _body(mask_ref, logits_ref, conv_ref, out_ref):
    logits = jnp.broadcast_to(logits_ref[...], (3 * NA, NCELL))
    conv = jnp.broadcast_to(conv_ref[...], (3 * NA, NCELL))
    cls = jax.lax.broadcasted_iota(jnp.int32, (3 * NA, NCELL), 0)
    onehot = conv == jnp.where(cls >= NA, jnp.where(cls >= 2 * NA, cls - 2 * NA,
                                                    cls - NA), cls)
    hi = logits.astype(jnp.bfloat16).astype(jnp.float32)
    val = jnp.where(cls < NA, hi, jnp.where(cls < 2 * NA, logits - hi, 1.0))
    wt = jnp.where(onehot, val, 0.0).astype(jnp.bfloat16)

    maskb = mask_ref[...].reshape(NCELL, BLK).astype(jnp.bfloat16)
    acc = jax.lax.dot_general(
        wt, maskb, (((1,), (0,)), ((), ())),
        preferred_element_type=jnp.float32)

    sums = acc[:NA, :] + acc[NA:2 * NA, :]
    counts = acc[2 * NA:, :]
    total = jnp.sum(counts, axis=0, keepdims=True)
    row = jax.lax.broadcasted_iota(jnp.int32, sums.shape, 0)
    scaled = jnp.where(row == NA - 1, sums * (1.0 / 225.0), sums)
    out = jnp.where(counts > 0.5, scaled, FMIN)
    out_ref[...] = jnp.where((total < 0.5) & (row == 0), 1.0, out)


def kernel(logits, monoaction_mask, monofield_base_converter):
    mask_t = monoaction_mask.transpose(1, 2, 0).view(jnp.int8)
    out_t = pl.pallas_call(
        _body,
        grid=(NB // BLK,),
        in_specs=[
            pl.BlockSpec((GRID[0], GRID[1], BLK), lambda i: (0, 0, i)),
            pl.BlockSpec((1, NCELL), lambda i: (0, 0)),
            pl.BlockSpec((1, NCELL), lambda i: (0, 0)),
        ],
        out_specs=pl.BlockSpec((NA, BLK), lambda i: (0, i)),
        out_shape=jax.ShapeDtypeStruct((NA, NB), jnp.float32),
    )(mask_t, logits.reshape(1, NCELL),
      monofield_base_converter.reshape(1, NCELL))
    return out_t.T
